# Initial kernel scaffold; baseline (speedup 1.0000x reference)
import jax
import jax.numpy as jnp
from jax.experimental import pallas as pl


def kernel(signal):
    raise NotImplementedError("write your pallas kernel here")


# speedup vs baseline: 98.2051x; 98.2051x over previous
import jax
import jax.numpy as jnp
from jax import lax
from jax.experimental import pallas as pl
from jax.experimental.pallas import tpu as pltpu
from jax.experimental.pallas import tpu_sc as plsc

B = 16
FRAMES = 32768
FLEN = 16
OUT_LEN = 2 * (FRAMES - 1) + FLEN
OUT_PAD = 65552
F = 2048
CHUNKS_PER_BATCH = FRAMES // F
NW = 32
CHUNKS_PER_W = (B * CHUNKS_PER_BATCH) // NW
IN_ROWS = F + 15
Q = F // 8


def _oa_body(sig_hbm, out_hbm, in_buf, out_buf):
    wid = lax.axis_index("s") * 2 + lax.axis_index("c")
    batch = wid // 2
    half = wid % 2

    iota = lax.iota(jnp.int32, 16)
    b_vec = (iota >> 1) * 16 + (iota & 1)
    zvec = jnp.zeros((16,), jnp.float32)

    for r in range(8):
        in_buf[pl.ds((F + 7 + r) * 16, 16)] = zvec

    for i in range(CHUNKS_PER_W):
        cib = half * CHUNKS_PER_W + i
        f0 = cib * F
        src = batch * (FRAMES * FLEN) + f0 * FLEN
        is_front = f0 == 0
        is_tail = cib == (CHUNKS_PER_BATCH - 1)

        pltpu.sync_copy(sig_hbm.at[pl.ds(src, F * FLEN)],
                        in_buf.at[pl.ds(7 * FLEN, F * FLEN)])
        @pl.when(jnp.logical_not(is_front))
        def _():
            pltpu.sync_copy(sig_hbm.at[pl.ds(src - 7 * FLEN, 7 * FLEN)],
                            in_buf.at[pl.ds(0, 7 * FLEN)])

        @pl.when(is_front)
        def _():
            for r in range(7):
                in_buf[pl.ds(r * 16, 16)] = zvec

        def compute_vec(q):
            base = 128 * q + 112
            acc = None
            for j in range(8):
                idx = b_vec + (base - 14 * j)
                g = plsc.load_gather(in_buf, [idx])
                acc = g if acc is None else acc + g
            return acc

        def body(q, _):
            out_buf[pl.ds(16 * q, 16)] = compute_vec(q)
            return 0

        lax.fori_loop(0, Q, body, 0)

        @pl.when(is_tail)
        def _():
            out_buf[pl.ds(16 * Q, 16)] = compute_vec(Q)

        dst = batch * OUT_PAD + 2 * f0
        pltpu.sync_copy(out_buf.at[pl.ds(0, 2 * F)],
                        out_hbm.at[pl.ds(dst, 2 * F)])

        @pl.when(is_tail)
        def _():
            pltpu.sync_copy(out_buf.at[pl.ds(2 * F, 16)],
                            out_hbm.at[pl.ds(dst + 2 * F, 16)])


_oa_kernel = pl.kernel(
    _oa_body,
    out_type=jax.ShapeDtypeStruct((B * OUT_PAD,), jnp.float32),
    mesh=plsc.VectorSubcoreMesh(core_axis_name="c", subcore_axis_name="s"),
    scratch_types=[
        pltpu.VMEM((IN_ROWS * FLEN,), jnp.float32),
        pltpu.VMEM((2 * F + 16,), jnp.float32),
    ],
    compiler_params=pltpu.CompilerParams(needs_layout_passes=False),
)


@jax.jit
def kernel(signal):
    flat = signal.reshape(B * FRAMES * FLEN)
    out = _oa_kernel(flat)
    return out.reshape(B, OUT_PAD)[:, :OUT_LEN]


# speedup vs baseline: 111.5497x; 1.1359x over previous
import jax
import jax.numpy as jnp
from jax import lax
from jax.experimental import pallas as pl
from jax.experimental.pallas import tpu as pltpu
from jax.experimental.pallas import tpu_sc as plsc

B = 16
FRAMES = 32768
FLEN = 16
SIG = FRAMES * FLEN
OUT_LEN = 2 * (FRAMES - 1) + FLEN
OUT_PAD = 65552
F = 2048
CHUNKS_PER_BATCH = FRAMES // F
CHUNKS_PER_W = CHUNKS_PER_BATCH // 2
IN_ROWS = F + 15
Q = F // 8


def _oa_body(sig_hbm, out_hbm, in0, in1, ob0, ob1, is0, is1, os0, os1):
    wid = lax.axis_index("s") * 2 + lax.axis_index("c")
    batch = wid // 2
    half = wid % 2

    in_bufs = (in0, in1)
    out_bufs = (ob0, ob1)
    in_sems = (is0, is1)
    out_sems = (os0, os1)

    iota = lax.iota(jnp.int32, 16)
    b_vec = (iota >> 1) * 16 + (iota & 1)
    bjs = [b_vec + (112 - 14 * j) for j in range(8)]
    zvec = jnp.zeros((16,), jnp.float32)

    for buf in in_bufs:
        for r in range(8):
            buf[pl.ds((F + 7 + r) * 16, 16)] = zvec

    def chunk_f0(i):
        return (half * CHUNKS_PER_W + i) * F

    def issue_in(i):
        f0 = chunk_f0(i)
        nb = i % 2
        src_h = batch * SIG + lax.max(f0 - 7, 0) * FLEN
        d_h = pltpu.async_copy(sig_hbm.at[pl.ds(src_h, 7 * FLEN)],
                               in_bufs[nb].at[pl.ds(0, 7 * FLEN)],
                               in_sems[nb])
        d_m = pltpu.async_copy(sig_hbm.at[pl.ds(batch * SIG + f0 * FLEN,
                                                F * FLEN)],
                               in_bufs[nb].at[pl.ds(7 * FLEN, F * FLEN)],
                               in_sems[nb])
        return d_h, d_m

    def compute_vec(buf, q):
        vq = jnp.full((16,), 128 * q, jnp.int32)
        acc = None
        for j in range(8):
            g = plsc.load_gather(buf, [bjs[j] + vq])
            acc = g if acc is None else acc + g
        return acc

    in_descs = issue_in(0)
    out_descs = [None] * CHUNKS_PER_W

    for i in range(CHUNKS_PER_W):
        nb = i % 2
        buf, ob = in_bufs[nb], out_bufs[nb]
        if i + 1 < CHUNKS_PER_W:
            next_descs = issue_in(i + 1)
        for d in in_descs:
            d.wait()
        in_descs = next_descs if i + 1 < CHUNKS_PER_W else ()

        if i == 0:
            @pl.when(half == 0)
            def _():
                for r in range(7):
                    buf[pl.ds(r * 16, 16)] = zvec

        if i >= 2:
            out_descs[i - 2].wait()

        @plsc.parallel_loop(0, Q, unroll=4)
        def _(q):
            ob[pl.ds(16 * q, 16)] = compute_vec(buf, q)

        if i == CHUNKS_PER_W - 1:
            @pl.when(half == 1)
            def _():
                ob[pl.ds(16 * Q, 16)] = compute_vec(buf, Q)

        dst = batch * OUT_PAD + 2 * chunk_f0(i)
        out_descs[i] = pltpu.async_copy(ob.at[pl.ds(0, 2 * F)],
                                        out_hbm.at[pl.ds(dst, 2 * F)],
                                        out_sems[nb])
        if i == CHUNKS_PER_W - 1:
            @pl.when(half == 1)
            def _():
                pltpu.sync_copy(ob.at[pl.ds(2 * F, 16)],
                                out_hbm.at[pl.ds(dst + 2 * F, 16)])

    out_descs[CHUNKS_PER_W - 2].wait()
    out_descs[CHUNKS_PER_W - 1].wait()


_oa_kernel = pl.kernel(
    _oa_body,
    out_type=jax.ShapeDtypeStruct((B * OUT_PAD,), jnp.float32),
    mesh=plsc.VectorSubcoreMesh(core_axis_name="c", subcore_axis_name="s"),
    scratch_types=[
        pltpu.VMEM((IN_ROWS * FLEN,), jnp.float32),
        pltpu.VMEM((IN_ROWS * FLEN,), jnp.float32),
        pltpu.VMEM((2 * F + 16,), jnp.float32),
        pltpu.VMEM((2 * F + 16,), jnp.float32),
        pltpu.SemaphoreType.DMA,
        pltpu.SemaphoreType.DMA,
        pltpu.SemaphoreType.DMA,
        pltpu.SemaphoreType.DMA,
    ],
    compiler_params=pltpu.CompilerParams(needs_layout_passes=False),
)


@jax.jit
def kernel(signal):
    flat = signal.reshape(B * SIG)
    out = _oa_kernel(flat)
    return out.reshape(B, OUT_PAD)[:, :OUT_LEN]


# speedup vs baseline: 112.5283x; 1.0088x over previous
import jax
import jax.numpy as jnp
from jax import lax
from jax.experimental import pallas as pl
from jax.experimental.pallas import tpu as pltpu
from jax.experimental.pallas import tpu_sc as plsc

B = 16
FRAMES = 32768
FLEN = 16
OUT_LEN = 2 * (FRAMES - 1) + FLEN
OUT_PAD = 65552
F = 2048
CHUNKS_PER_BATCH = FRAMES // F
CHUNKS_PER_W = CHUNKS_PER_BATCH // 2
IN_ROWS = F + 15
Q = F // 8


def _oa_body(sig_hbm, out_hbm, in0, in1, ob0, ob1, is0, is1, os0, os1):
    wid = lax.axis_index("s") * 2 + lax.axis_index("c")
    batch = wid // 2
    half = wid % 2

    in_bufs = (in0, in1)
    out_bufs = (ob0, ob1)
    in_sems = (is0, is1)
    out_sems = (os0, os1)

    iota = lax.iota(jnp.int32, 16)
    rows_j = [(iota >> 1) + (7 - j) for j in range(8)]
    cols_j = [(iota & 1) + 2 * j for j in range(8)]
    zvec = jnp.zeros((16,), jnp.float32)

    for buf in in_bufs:
        for r in range(8):
            buf[F + 7 + r, :] = zvec

    def chunk_f0(i):
        return (half * CHUNKS_PER_W + i) * F

    def issue_in(i):
        f0 = chunk_f0(i)
        nb = i % 2
        d_h = pltpu.async_copy(
            sig_hbm.at[batch, pl.ds(lax.max(f0 - 7, 0), 7), :],
            in_bufs[nb].at[pl.ds(0, 7), :], in_sems[nb])
        d_m = pltpu.async_copy(
            sig_hbm.at[batch, pl.ds(f0, F), :],
            in_bufs[nb].at[pl.ds(7, F), :], in_sems[nb])
        return d_h, d_m

    def compute_vec(buf, q):
        vq = jnp.full((16,), 8 * q, jnp.int32)
        acc = None
        for j in range(8):
            g = plsc.load_gather(buf, [rows_j[j] + vq, cols_j[j]])
            acc = g if acc is None else acc + g
        return acc

    in_descs = issue_in(0)
    out_descs = [None] * CHUNKS_PER_W

    for i in range(CHUNKS_PER_W):
        nb = i % 2
        buf, ob = in_bufs[nb], out_bufs[nb]
        if i + 1 < CHUNKS_PER_W:
            next_descs = issue_in(i + 1)
        for d in in_descs:
            d.wait()
        in_descs = next_descs if i + 1 < CHUNKS_PER_W else ()

        if i == 0:
            @pl.when(half == 0)
            def _():
                for r in range(7):
                    buf[r, :] = zvec

        if i >= 2:
            out_descs[i - 2].wait()

        @plsc.parallel_loop(0, Q, unroll=4)
        def _(q):
            ob[pl.ds(16 * q, 16)] = compute_vec(buf, q)

        if i == CHUNKS_PER_W - 1:
            @pl.when(half == 1)
            def _():
                ob[pl.ds(16 * Q, 16)] = compute_vec(buf, Q)

        dst = batch * OUT_PAD + 2 * chunk_f0(i)
        out_descs[i] = pltpu.async_copy(ob.at[pl.ds(0, 2 * F)],
                                        out_hbm.at[pl.ds(dst, 2 * F)],
                                        out_sems[nb])
        if i == CHUNKS_PER_W - 1:
            @pl.when(half == 1)
            def _():
                pltpu.sync_copy(ob.at[pl.ds(2 * F, 16)],
                                out_hbm.at[pl.ds(dst + 2 * F, 16)])

    out_descs[CHUNKS_PER_W - 2].wait()
    out_descs[CHUNKS_PER_W - 1].wait()


_oa_kernel = pl.kernel(
    _oa_body,
    out_type=jax.ShapeDtypeStruct((B * OUT_PAD,), jnp.float32),
    mesh=plsc.VectorSubcoreMesh(core_axis_name="c", subcore_axis_name="s"),
    scratch_types=[
        pltpu.VMEM((IN_ROWS, FLEN), jnp.float32),
        pltpu.VMEM((IN_ROWS, FLEN), jnp.float32),
        pltpu.VMEM((2 * F + 16,), jnp.float32),
        pltpu.VMEM((2 * F + 16,), jnp.float32),
        pltpu.SemaphoreType.DMA,
        pltpu.SemaphoreType.DMA,
        pltpu.SemaphoreType.DMA,
        pltpu.SemaphoreType.DMA,
    ],
    compiler_params=pltpu.CompilerParams(needs_layout_passes=False,
                                         use_tc_tiling_on_sc=False),
)


@jax.jit
def kernel(signal):
    out = _oa_kernel(signal)
    return out.reshape(B, OUT_PAD)[:, :OUT_LEN]


# speedup vs baseline: 514.7092x; 4.5740x over previous
import jax
import jax.numpy as jnp
from jax import lax
from jax.experimental import pallas as pl
from jax.experimental.pallas import tpu as pltpu
from jax.experimental.pallas import tpu_sc as plsc

B = 16
FRAMES = 32768
FLEN = 16
OUT_LEN = 2 * (FRAMES - 1) + FLEN
F = 2048
CPB = FRAMES // F
CPW = CPB // 2
NT = 18
KTSZ = 256 * 8 * 128
IN_SZ = 2 * NT * 1024
OT = 513
NBUF = 3


def _oa_body(s5_hbm, o4_hbm, in0, in1, in2, ob0, ob1,
             is0, is1, is2, os0, os1):
    wid = lax.axis_index("s") * 2 + lax.axis_index("c")
    batch = wid // 2
    half = wid % 2
    bt = batch // 8
    bp = batch % 8

    in_bufs = (in0, in1, in2)
    out_bufs = (ob0, ob1)
    in_sems = (is0, is1, is2)
    out_sems = (os0, os1)

    iota = lax.iota(jnp.int32, 16)
    phi = (iota >> 1) + 128 * (iota & 1)
    phi_j = [phi] + [phi + jnp.where((iota >> 1) < j, -896, 0)
                     for j in range(1, 8)]
    zvec = jnp.zeros((16,), jnp.float32)

    def chunk_f0(i):
        return (half * CPW + i) * F

    def issue_in(i):
        f0 = chunk_f0(i)
        nb = i % NBUF
        descs = []
        for kt in range(2):
            src = (batch * 2 + kt) * KTSZ
            descs.append(pltpu.async_copy(
                s5_hbm.at[pl.ds(src + (f0 // 128) * 1024, 16 * 1024)],
                in_bufs[nb].at[pl.ds(kt * NT * 1024 + 1024, 16 * 1024)],
                in_sems[nb]))
            descs.append(pltpu.async_copy(
                s5_hbm.at[pl.ds(src + lax.max(f0 // 128 - 1, 0) * 1024,
                                1024)],
                in_bufs[nb].at[pl.ds(kt * NT * 1024, 1024)],
                in_sems[nb]))
        return descs

    def compute_group(buf, rp_plus1, rhalf, c0):
        fpb = 64 * rhalf + c0 // 2
        acc = None
        for j in range(8):
            const = 17408 * (j >= 4) + 256 * j + fpb - j
            vec = phi_j[j] if fpb == 0 else phi
            g = plsc.load_gather(buf, [vec + (1024 * rp_plus1 + const)])
            acc = g if acc is None else acc + g
        return acc

    in_descs = [issue_in(i) for i in range(NBUF)]

    zb = in_bufs[(CPW - 1) % NBUF]
    for kt in range(2):
        for v in range(64):
            zb[pl.ds(kt * NT * 1024 + 17 * 1024 + 16 * v, 16)] = zvec

    out_descs = [None] * CPW

    for i in range(CPW):
        nb = i % NBUF
        buf, ob = in_bufs[nb], out_bufs[i % 2]
        for d in in_descs[i]:
            d.wait()

        if i == 0:
            @pl.when(half == 0)
            def _():
                for kt in range(2):
                    for v in range(64):
                        buf[pl.ds(kt * NT * 1024 + 16 * v, 16)] = zvec

        if i >= 2:
            out_descs[i - 2].wait()

        @plsc.parallel_loop(0, 16)
        def _(rp):
            for rhalf in range(2):
                r = 2 * rp + rhalf
                for c0 in range(0, 128, 16):
                    ob[r, pl.ds(c0, 16)] = compute_group(
                        buf, rp + 1, rhalf, c0)

        if i == CPW - 1:
            @pl.when(half == 1)
            def _():
                for c0 in range(0, 128, 16):
                    ob[32, pl.ds(c0, 16)] = compute_group(buf, 17, 0, c0)

        cib = half * CPW + i
        out_descs[i] = pltpu.async_copy(
            ob.at[pl.ds(0, 32), :],
            o4_hbm.at[bt, pl.ds(32 * cib, 32), bp, :],
            out_sems[i % 2])
        if i == CPW - 1:
            @pl.when(half == 1)
            def _():
                pltpu.sync_copy(ob.at[32, :], o4_hbm.at[bt, 512, bp, :])

        if i + NBUF < CPW:
            in_descs.append(issue_in(i + NBUF))

    out_descs[CPW - 2].wait()
    out_descs[CPW - 1].wait()


_oa_kernel = pl.kernel(
    _oa_body,
    out_type=jax.ShapeDtypeStruct((2, OT, 8, 128), jnp.float32),
    mesh=plsc.VectorSubcoreMesh(core_axis_name="c", subcore_axis_name="s"),
    scratch_types=[
        pltpu.VMEM((IN_SZ,), jnp.float32),
        pltpu.VMEM((IN_SZ,), jnp.float32),
        pltpu.VMEM((IN_SZ,), jnp.float32),
        pltpu.VMEM((33, 128), jnp.float32),
        pltpu.VMEM((33, 128), jnp.float32),
        pltpu.SemaphoreType.DMA,
        pltpu.SemaphoreType.DMA,
        pltpu.SemaphoreType.DMA,
        pltpu.SemaphoreType.DMA,
        pltpu.SemaphoreType.DMA,
    ],
    compiler_params=pltpu.CompilerParams(needs_layout_passes=False,
                                         use_tc_tiling_on_sc=False),
)


@jax.jit
def kernel(signal):
    s5 = signal.reshape(B, 256, 128, 2, 8).transpose(0, 3, 1, 4, 2)
    o4 = _oa_kernel(s5.reshape(-1))
    return o4.transpose(0, 2, 1, 3).reshape(B, OT * 128)[:, :OUT_LEN]


# speedup vs baseline: 526.5708x; 1.0230x over previous
import jax
import jax.numpy as jnp
from jax import lax
from jax.experimental import pallas as pl
from jax.experimental.pallas import tpu as pltpu
from jax.experimental.pallas import tpu_sc as plsc

B = 16
FRAMES = 32768
FLEN = 16
OUT_LEN = 2 * (FRAMES - 1) + FLEN
F = 2048
CPB = FRAMES // F
CPW = CPB // 2
NT = 18
KTSZ = 256 * 8 * 128
IN_SZ = 2 * NT * 1024
OT = 513
NBUF = 2


def _oa_body(s5_hbm, o4_hbm, in0, in1, ob0, ob1, is0, is1, os0, os1):
    wid = lax.axis_index("s") * 2 + lax.axis_index("c")
    batch = wid // 2
    half = wid % 2
    bt = batch // 8
    bp = batch % 8

    in_bufs = (in0, in1)
    out_bufs = (ob0, ob1)
    in_sems = (is0, is1)
    out_sems = (os0, os1)

    iota = lax.iota(jnp.int32, 16)
    phi = (iota >> 1) + 128 * (iota & 1)
    phi_j = [phi] + [phi + jnp.where((iota >> 1) < j, -896, 0)
                     for j in range(1, 8)]
    zvec = jnp.zeros((16,), jnp.float32)

    def chunk_f0(i):
        return (half * CPW + i) * F

    def issue_in(i):
        f0 = chunk_f0(i)
        nb = i % NBUF
        descs = []
        if i == 0:
            for kt in range(2):
                src = (batch * 2 + kt) * KTSZ
                descs.append(pltpu.async_copy(
                    s5_hbm.at[pl.ds(src + (f0 // 128) * 1024, 16 * 1024)],
                    in_bufs[nb].at[pl.ds(kt * NT * 1024 + 1024, 16 * 1024)],
                    in_sems[nb]))
                descs.append(pltpu.async_copy(
                    s5_hbm.at[pl.ds(src + lax.max(f0 // 128 - 1, 0) * 1024,
                                    1024)],
                    in_bufs[nb].at[pl.ds(kt * NT * 1024, 1024)],
                    in_sems[nb]))
        else:
            for kt in range(2):
                src = (batch * 2 + kt) * KTSZ + (f0 // 128 - 1) * 1024
                descs.append(pltpu.async_copy(
                    s5_hbm.at[pl.ds(src, 17 * 1024)],
                    in_bufs[nb].at[pl.ds(kt * NT * 1024, 17 * 1024)],
                    in_sems[nb]))
        return descs

    def compute_group(buf, rp_plus1, rhalf, c0):
        fpb = 64 * rhalf + c0 // 2
        acc = None
        for j in range(8):
            const = 17408 * (j >= 4) + 256 * j + fpb - j
            vec = phi_j[j] if fpb == 0 else phi
            g = plsc.load_gather(buf, [vec + (1024 * rp_plus1 + const)])
            acc = g if acc is None else acc + g
        return acc

    in_descs = [issue_in(i) for i in range(NBUF)]

    zb = in_bufs[(CPW - 1) % NBUF]
    for kt in range(2):
        for v in range(64):
            zb[pl.ds(kt * NT * 1024 + 17 * 1024 + 16 * v, 16)] = zvec

    out_descs = [None] * CPW

    for i in range(CPW):
        nb = i % NBUF
        buf, ob = in_bufs[nb], out_bufs[i % 2]
        for d in in_descs[i]:
            d.wait()

        if i == 0:
            @pl.when(half == 0)
            def _():
                for kt in range(2):
                    for v in range(64):
                        buf[pl.ds(kt * NT * 1024 + 16 * v, 16)] = zvec

        if i >= 2:
            out_descs[i - 2].wait()

        @plsc.parallel_loop(0, 16)
        def _(rp):
            for rhalf in range(2):
                r = 2 * rp + rhalf
                for c0 in range(0, 128, 16):
                    ob[r, pl.ds(c0, 16)] = compute_group(
                        buf, rp + 1, rhalf, c0)

        if i == CPW - 1:
            @pl.when(half == 1)
            def _():
                for c0 in range(0, 128, 16):
                    ob[32, pl.ds(c0, 16)] = compute_group(buf, 17, 0, c0)

        cib = half * CPW + i
        out_descs[i] = pltpu.async_copy(
            ob.at[pl.ds(0, 32), :],
            o4_hbm.at[bt, pl.ds(32 * cib, 32), bp, :],
            out_sems[i % 2])
        if i == CPW - 1:
            @pl.when(half == 1)
            def _():
                pltpu.sync_copy(ob.at[32, :], o4_hbm.at[bt, 512, bp, :])

        if i + NBUF < CPW:
            in_descs.append(issue_in(i + NBUF))

    out_descs[CPW - 2].wait()
    out_descs[CPW - 1].wait()


_oa_kernel = pl.kernel(
    _oa_body,
    out_type=jax.ShapeDtypeStruct((2, OT, 8, 128), jnp.float32),
    mesh=plsc.VectorSubcoreMesh(core_axis_name="c", subcore_axis_name="s"),
    scratch_types=[
        pltpu.VMEM((IN_SZ,), jnp.float32),
        pltpu.VMEM((IN_SZ,), jnp.float32),
        pltpu.VMEM((33, 128), jnp.float32),
        pltpu.VMEM((33, 128), jnp.float32),
        pltpu.SemaphoreType.DMA,
        pltpu.SemaphoreType.DMA,
        pltpu.SemaphoreType.DMA,
        pltpu.SemaphoreType.DMA,
    ],
    compiler_params=pltpu.CompilerParams(needs_layout_passes=False,
                                         use_tc_tiling_on_sc=False),
)


@jax.jit
def kernel(signal):
    s5 = signal.reshape(B, 256, 128, 2, 8).transpose(0, 3, 1, 4, 2)
    o4 = _oa_kernel(s5.reshape(-1))
    return o4.transpose(0, 2, 1, 3).reshape(B, OT * 128)[:, :OUT_LEN]
